# baseline (speedup 1.0000x reference)
import functools

import jax
import jax.numpy as jnp
from jax import lax
from jax.experimental import pallas as pl
from jax.experimental.pallas import tpu as pltpu
from jax.experimental.pallas import tpu_sc as plsc

DIM = 128
NC = 2
NS = 16
NW = NC * NS
CH = 128
GPB = 2
BC = CH * GPB
NBUF = 3


@functools.lru_cache(maxsize=None)
def _build(B, V):
    BPW = B // NW
    NCH = BPW // CH
    NBC = BPW // BC
    mesh = plsc.VectorSubcoreMesh(core_axis_name="c", subcore_axis_name="s")

    @functools.partial(
        pl.kernel,
        mesh=mesh,
        out_type=jax.ShapeDtypeStruct((B, DIM), jnp.float32),
        scratch_types=[
            pltpu.VMEM((NCH, CH), jnp.int32),
            *[pltpu.VMEM((BC, DIM), jnp.float32) for _ in range(NBUF)],
            *[pltpu.SemaphoreType.DMA for _ in range(2 * NBUF)],
        ],
    )
    def emb(idx_hbm, table_hbm, out_hbm, idx_v, *rest):
        bufs = rest[:NBUF]
        gsems = rest[NBUF:2 * NBUF]
        osems = rest[2 * NBUF:]
        wid = lax.axis_index("s") * NC + lax.axis_index("c")
        base = wid * BPW
        pltpu.sync_copy(idx_hbm.at[wid], idx_v)

        def gather(t, h, b):
            return pltpu.make_async_copy(
                table_hbm.at[idx_v.at[GPB * t + h]],
                bufs[b].at[pl.ds(h * CH, CH)], gsems[b])

        def gstart(t, b):
            for h in range(GPB):
                gather(t, h, b).start()

        def gwait(t, b):
            for h in range(GPB):
                gather(t, h, b).wait()

        def outcp(t, b):
            return pltpu.make_async_copy(
                bufs[b], out_hbm.at[pl.ds(base + t * BC, BC)], osems[b])

        G = NBC // NBUF

        for b in range(NBUF):
            gstart(b, b)

        for t in range(NBUF):
            gwait(t, t)
            outcp(t, t).start()
            if t >= 1:
                outcp(t - 1, t - 1).wait()
                gstart(t + NBUF - 1, t - 1)

        def body(g, carry):
            for b in range(NBUF):
                t = g * NBUF + b
                gwait(t, b)
                outcp(t, b).start()
                pb = (b - 1) % NBUF
                outcp(t - 1, pb).wait()
                gstart(t + NBUF - 1, pb)
            return carry

        lax.fori_loop(1, G - 1, body, 0)

        for t in range(NBUF * (G - 1), NBC):
            b = t % NBUF
            gwait(t, b)
            outcp(t, b).start()
            pb = (t - 1) % NBUF
            outcp(t - 1, pb).wait()
            if t + NBUF - 1 <= NBC - 1:
                gstart(t + NBUF - 1, pb)
        outcp(NBC - 1, (NBC - 1) % NBUF).wait()

    return emb


def kernel(x, table):
    S0, S1 = x.shape
    B = S0 * S1
    idx = x.reshape(NW, B // NW // CH, CH).astype(jnp.int32)
    out = _build(B, table.shape[0])(idx, table)
    return out.reshape(S0, S1, DIM)


# speedup vs baseline: 1.0038x; 1.0038x over previous
import functools

import jax
import jax.numpy as jnp
from jax import lax
from jax.experimental import pallas as pl
from jax.experimental.pallas import tpu as pltpu
from jax.experimental.pallas import tpu_sc as plsc

DIM = 128
NC = 2
NS = 16
NW = NC * NS
CH = 128
GPB = 1
BC = CH * GPB
NBUF = 6


@functools.lru_cache(maxsize=None)
def _build(B, V):
    BPW = B // NW
    NCH = BPW // CH
    NBC = BPW // BC
    mesh = plsc.VectorSubcoreMesh(core_axis_name="c", subcore_axis_name="s")

    @functools.partial(
        pl.kernel,
        mesh=mesh,
        out_type=jax.ShapeDtypeStruct((B, DIM), jnp.float32),
        scratch_types=[
            pltpu.VMEM((NCH, CH), jnp.int32),
            *[pltpu.VMEM((BC, DIM), jnp.float32) for _ in range(NBUF)],
            *[pltpu.SemaphoreType.DMA for _ in range(2 * NBUF)],
        ],
    )
    def emb(idx_hbm, table_hbm, out_hbm, idx_v, *rest):
        bufs = rest[:NBUF]
        gsems = rest[NBUF:2 * NBUF]
        osems = rest[2 * NBUF:]
        wid = lax.axis_index("s") * NC + lax.axis_index("c")
        base = wid * BPW
        pltpu.sync_copy(idx_hbm.at[wid], idx_v)

        def gather(t, h, b):
            return pltpu.make_async_copy(
                table_hbm.at[idx_v.at[GPB * t + h]],
                bufs[b].at[pl.ds(h * CH, CH)], gsems[b])

        def gstart(t, b):
            for h in range(GPB):
                gather(t, h, b).start()

        def gwait(t, b):
            for h in range(GPB):
                gather(t, h, b).wait()

        def outcp(t, b):
            return pltpu.make_async_copy(
                bufs[b], out_hbm.at[pl.ds(base + t * BC, BC)], osems[b])

        G = NBC // NBUF

        for b in range(NBUF):
            gstart(b, b)

        for t in range(NBUF):
            gwait(t, t)
            outcp(t, t).start()
            if t >= 1:
                outcp(t - 1, t - 1).wait()
                gstart(t + NBUF - 1, t - 1)

        def body(g, carry):
            for b in range(NBUF):
                t = g * NBUF + b
                gwait(t, b)
                outcp(t, b).start()
                pb = (b - 1) % NBUF
                outcp(t - 1, pb).wait()
                gstart(t + NBUF - 1, pb)
            return carry

        lax.fori_loop(1, G - 1, body, 0)

        for t in range(NBUF * (G - 1), NBC):
            b = t % NBUF
            gwait(t, b)
            outcp(t, b).start()
            pb = (t - 1) % NBUF
            outcp(t - 1, pb).wait()
            if t + NBUF - 1 <= NBC - 1:
                gstart(t + NBUF - 1, pb)
        outcp(NBC - 1, (NBC - 1) % NBUF).wait()

    return emb


def kernel(x, table):
    S0, S1 = x.shape
    B = S0 * S1
    idx = x.reshape(NW, B // NW // CH, CH).astype(jnp.int32)
    out = _build(B, table.shape[0])(idx, table)
    return out.reshape(S0, S1, DIM)
